# baseline (speedup 1.0000x reference)
import functools

import jax
import jax.numpy as jnp
from jax import lax
from jax.experimental import pallas as pl
from jax.experimental.pallas import tpu as pltpu
from jax.experimental.pallas import tpu_sc as plsc

N = 10000
E = 320000
DN = 128
DE = 16
H = 32

NC = 2
NS = 16
NW = NC * NS
L = 16

PER_TILE = E // NW
CH = 80
N_CHUNKS = PER_TILE // CH
GRP = 16


def _ab_body(x_ref, w_ref, o_ref):
    o_ref[...] = jnp.dot(x_ref[...], w_ref[...],
                         preferred_element_type=jnp.float32)


def _edge_c_body(ea_ref, eat_ref, wa_ref, wb_ref, b_ref,
                 c1a_ref, c1b_ref, c2a_ref, c2b_ref):
    wa = wa_ref[...]
    wb = wb_ref[...]
    b = b_ref[...]
    ea = ea_ref[...]
    eat = eat_ref[...]
    c1a_ref[...] = jnp.dot(ea, wa, preferred_element_type=jnp.float32) + b
    c1b_ref[...] = jnp.dot(ea, wb, preferred_element_type=jnp.float32) + b
    c2a_ref[...] = jnp.dot(eat, wa, preferred_element_type=jnp.float32) + b
    c2b_ref[...] = jnp.dot(eat, wb, preferred_element_type=jnp.float32) + b


def _tc_precompute(x, edge_attr, edge_attr_T, W1, b1):
    w_ab = jnp.concatenate([W1[:DN], W1[DN:2 * DN]], axis=1)
    ab = pl.pallas_call(
        _ab_body,
        grid=(10,),
        in_specs=[
            pl.BlockSpec((N // 10, DN), lambda i: (i, 0)),
            pl.BlockSpec((DN, 2 * H), lambda i: (0, 0)),
        ],
        out_specs=pl.BlockSpec((N // 10, 2 * H), lambda i: (i, 0)),
        out_shape=jax.ShapeDtypeStruct((N, 2 * H), jnp.float32),
    )(x, w_ab)

    w1c = W1[2 * DN:]
    bd = jnp.kron(jnp.eye(4, dtype=jnp.float32), w1c)
    z = jnp.zeros((64, 128), jnp.float32)
    wa = jnp.concatenate([bd, z], axis=0)
    wb = jnp.concatenate([z, bd], axis=0)
    b4 = jnp.tile(b1, 4).reshape(1, 128)

    ea_r, eat_r = _sc_compact(edge_attr, edge_attr_T)
    G = E // 8
    BG = G // 40
    spec = pl.BlockSpec((BG, 128), lambda i: (i, 0))
    wspec = pl.BlockSpec((128, 128), lambda i: (0, 0))
    cshape = jax.ShapeDtypeStruct((G, 128), jnp.float32)
    c1a, c1b, c2a, c2b = pl.pallas_call(
        _edge_c_body,
        grid=(40,),
        in_specs=[spec, spec, wspec, wspec,
                  pl.BlockSpec((1, 128), lambda i: (0, 0))],
        out_specs=[spec, spec, spec, spec],
        out_shape=[cshape, cshape, cshape, cshape],
    )(ea_r, eat_r, wa, wb, b4)
    return ab, c1a, c1b, c2a, c2b


RCH = 512
N_RCH = E // RCH


@functools.partial(
    pl.kernel,
    out_type=[jax.ShapeDtypeStruct((E // 8, 128), jnp.float32),
              jax.ShapeDtypeStruct((E // 8, 128), jnp.float32)],
    mesh=plsc.VectorSubcoreMesh(core_axis_name="c", subcore_axis_name="s"),
    compiler_params=pltpu.CompilerParams(needs_layout_passes=False,
                                         use_tc_tiling_on_sc=True),
    scratch_types=[
        pltpu.VMEM((RCH, DE), jnp.float32),
        pltpu.VMEM((RCH // 8, 128), jnp.float32),
    ],
)
def _sc_compact(ea_hbm, eat_hbm, o1_hbm, o2_hbm, inbuf, outbuf):
    wid = lax.axis_index("s") * NC + lax.axis_index("c")

    def one_array(src_hbm, dst_hbm):
        def chunk(k, carry):
            c = wid + k * NW

            @pl.when(c < N_RCH)
            def _():
                r0 = pl.multiple_of(c * RCH, RCH)
                pltpu.sync_copy(src_hbm.at[pl.ds(r0, RCH), :], inbuf)

                def rows(blk, carry2):
                    for jj in range(8):
                        v = inbuf[blk * 8 + jj, pl.ds(0, DE)]
                        outbuf[blk, pl.ds(jj * DE, DE)] = v
                    return carry2

                lax.fori_loop(0, RCH // 8, rows, 0, unroll=False)
                pltpu.sync_copy(
                    outbuf,
                    dst_hbm.at[pl.ds(pl.multiple_of(c * (RCH // 8), RCH // 8),
                                     RCH // 8), :])

            return carry

        lax.fori_loop(0, (N_RCH + NW - 1) // NW, chunk, 0, unroll=False)

    one_array(ea_hbm, o1_hbm)
    one_array(eat_hbm, o2_hbm)


def _buf_set():
    return [
        pltpu.VMEM((CH, 2 * H), jnp.float32),
        pltpu.VMEM((CH, 2 * H), jnp.float32),
        pltpu.VMEM((CH // 8, 128), jnp.float32),
        pltpu.VMEM((CH // 8, 128), jnp.float32),
        pltpu.VMEM((CH // 8, 128), jnp.float32),
        pltpu.VMEM((CH // 8, 128), jnp.float32),
    ]


@functools.partial(
    pl.kernel,
    out_type=jax.ShapeDtypeStruct((E,), jnp.float32),
    mesh=plsc.VectorSubcoreMesh(core_axis_name="c", subcore_axis_name="s"),
    compiler_params=pltpu.CompilerParams(needs_layout_passes=False,
                                         use_tc_tiling_on_sc=False),
    scratch_types=[
        pltpu.VMEM((PER_TILE,), jnp.int32),
        pltpu.VMEM((PER_TILE,), jnp.int32),
        *_buf_set(),
        *_buf_set(),
        pltpu.VMEM((3 * L,), jnp.float32),
        pltpu.VMEM((GRP * L,), jnp.float32),
        pltpu.VMEM((PER_TILE,), jnp.float32),
        pltpu.SemaphoreType.DMA,
        pltpu.SemaphoreType.DMA,
        pltpu.SemaphoreType.DMA,
        pltpu.SemaphoreType.DMA,
    ],
)
def _sc_edge_mlp(ab_hbm, c1a_hbm, c1b_hbm, c2a_hbm, c2b_hbm,
                 src_hbm, tgt_hbm, par_hbm, out_hbm,
                 srcv, tgtv,
                 absv_a, abtv_a, c1av_a, c1bv_a, c2av_a, c2bv_a,
                 absv_b, abtv_b, c1av_b, c1bv_b, c2av_b, c2bv_b,
                 pv, stg, outv,
                 sem_g_a, sem_g_b, sem_c_a, sem_c_b):
    wid = lax.axis_index("s") * NC + lax.axis_index("c")
    base = wid * PER_TILE

    pltpu.sync_copy(par_hbm, pv)
    pltpu.sync_copy(src_hbm.at[pl.ds(base, PER_TILE)], srcv)
    pltpu.sync_copy(tgt_hbm.at[pl.ds(base, PER_TILE)], tgtv)
    wd0 = pv[pl.ds(0, L)]
    wd1 = pv[pl.ds(L, L)]
    cdv = pv[pl.ds(2 * L, L)]
    col = lax.iota(jnp.int32, L) * L

    rings = (
        (absv_a, abtv_a, c1av_a, c1bv_a, c2av_a, c2bv_a, sem_g_a, sem_c_a),
        (absv_b, abtv_b, c1av_b, c1bv_b, c2av_b, c2bv_b, sem_g_b, sem_c_b),
    )

    def dma_set(k, ring):
        absv, abtv, c1av, c1bv, c2av, c2bv, sem_g, sem_c = ring
        koff = k * CH
        gr = (base + koff) // 8
        return (
            pltpu.make_async_copy(ab_hbm.at[srcv.at[pl.ds(koff, CH)]],
                                  absv, sem_g),
            pltpu.make_async_copy(ab_hbm.at[tgtv.at[pl.ds(koff, CH)]],
                                  abtv, sem_g),
            pltpu.make_async_copy(c1a_hbm.at[pl.ds(gr, CH // 8), :],
                                  c1av, sem_c),
            pltpu.make_async_copy(c1b_hbm.at[pl.ds(gr, CH // 8), :],
                                  c1bv, sem_c),
            pltpu.make_async_copy(c2a_hbm.at[pl.ds(gr, CH // 8), :],
                                  c2av, sem_c),
            pltpu.make_async_copy(c2b_hbm.at[pl.ds(gr, CH // 8), :],
                                  c2bv, sem_c),
        )

    def issue(k, ring):
        for cp in dma_set(k, ring):
            cp.start()

    def wait(k, ring):
        for cp in dma_set(k, ring):
            cp.wait()

    def compute(k, ring):
        absv, abtv, c1av, c1bv, c2av, c2bv, _, _ = ring
        koff = k * CH

        def grp_body(blk, carry2):
            e0 = blk * GRP
            for j in range(GRP):
                e = e0 + j
                el = j % 8
                gi = blk * 2 + j // 8
                if el < 4:
                    r1, r2, co = c1av, c2av, el * 32
                else:
                    r1, r2, co = c1bv, c2bv, (el - 4) * 32
                s0 = absv[e, pl.ds(0, L)]
                s1 = absv[e, pl.ds(L, L)]
                s2 = absv[e, pl.ds(2 * L, L)]
                s3 = absv[e, pl.ds(3 * L, L)]
                t0 = abtv[e, pl.ds(0, L)]
                t1 = abtv[e, pl.ds(L, L)]
                t2 = abtv[e, pl.ds(2 * L, L)]
                t3 = abtv[e, pl.ds(3 * L, L)]
                c10 = r1[gi, pl.ds(co, L)]
                c11 = r1[gi, pl.ds(co + L, L)]
                c20 = r2[gi, pl.ds(co, L)]
                c21 = r2[gi, pl.ds(co + L, L)]
                zero = jnp.zeros((L,), jnp.float32)
                h1a = jnp.maximum(s0 + t2 + c10, zero)
                h1b = jnp.maximum(s1 + t3 + c11, zero)
                h2a = jnp.maximum(t0 + s2 + c20, zero)
                h2b = jnp.maximum(t1 + s3 + c21, zero)
                t = (h1a + h2a) * wd0 + (h1b + h2b) * wd1
                stg[pl.ds(j * L, L)] = t
            sums = plsc.load_gather(stg, [col])
            for kk in range(1, L):
                sums = sums + plsc.load_gather(stg, [col + kk])
            z = sums + cdv
            outv[pl.ds(koff + e0, L)] = 1.0 / (1.0 + jnp.exp(-z))
            return carry2

        lax.fori_loop(0, CH // GRP, grp_body, 0, unroll=False)

    issue(0, rings[0])

    def pair_body(kp, carry):
        k0 = kp * 2
        issue(k0 + 1, rings[1])
        wait(k0, rings[0])
        compute(k0, rings[0])
        issue(k0 + 2, rings[0])
        wait(k0 + 1, rings[1])
        compute(k0 + 1, rings[1])
        return carry

    lax.fori_loop(0, (N_CHUNKS - 1) // 2, pair_body, 0, unroll=False)
    wait(N_CHUNKS - 1, rings[0])
    compute(N_CHUNKS - 1, rings[0])

    pltpu.sync_copy(outv, out_hbm.at[pl.ds(base, PER_TILE)])


def kernel(x, edge_index, edge_attr, edge_attr_T, W1, b1, W2, b2):
    ab, c1a, c1b, c2a, c2b = _tc_precompute(x, edge_attr, edge_attr_T, W1, b1)
    wd = 0.5 * (W2[:, 1] - W2[:, 0])
    cd = b2[1] - b2[0]
    params = jnp.concatenate([wd, jnp.full((L,), cd, jnp.float32)])
    src = edge_index[0].astype(jnp.int32)
    tgt = edge_index[1].astype(jnp.int32)
    out = _sc_edge_mlp(ab, c1a, c1b, c2a, c2b, src, tgt, params)
    return out.reshape(E, 1)


# speedup vs baseline: 1.3020x; 1.3020x over previous
import functools

import jax
import jax.numpy as jnp
from jax import lax
from jax.experimental import pallas as pl
from jax.experimental.pallas import tpu as pltpu
from jax.experimental.pallas import tpu_sc as plsc

N = 10000
E = 320000
DN = 128
DE = 16
H = 32

NC = 2
NS = 16
NW = NC * NS
L = 16

E0 = 161280
E1 = E - E0
CH = 80
GRP = 16


def _ab_body(x_ref, w_ref, o_ref):
    o_ref[...] = jnp.dot(x_ref[...], w_ref[...],
                         preferred_element_type=jnp.float32)


def _edge_c_body(ea_ref, eat_ref, wa_ref, wb_ref, b_ref,
                 c1a_ref, c1b_ref, c2a_ref, c2b_ref):
    wa = wa_ref[...]
    wb = wb_ref[...]
    b = b_ref[...]
    ea = ea_ref[...]
    eat = eat_ref[...]
    c1a_ref[...] = jnp.dot(ea, wa, preferred_element_type=jnp.float32) + b
    c1b_ref[...] = jnp.dot(ea, wb, preferred_element_type=jnp.float32) + b
    c2a_ref[...] = jnp.dot(eat, wa, preferred_element_type=jnp.float32) + b
    c2b_ref[...] = jnp.dot(eat, wb, preferred_element_type=jnp.float32) + b


def _ab_table(x, W1):
    w_ab = jnp.concatenate([W1[:DN], W1[DN:2 * DN]], axis=1)
    return pl.pallas_call(
        _ab_body,
        grid=(10,),
        in_specs=[
            pl.BlockSpec((N // 10, DN), lambda i: (i, 0)),
            pl.BlockSpec((DN, 2 * H), lambda i: (0, 0)),
        ],
        out_specs=pl.BlockSpec((N // 10, 2 * H), lambda i: (i, 0)),
        out_shape=jax.ShapeDtypeStruct((N, 2 * H), jnp.float32),
    )(x, w_ab)


def _edge_c(ea_r, eat_r, W1, b1):
    w1c = W1[2 * DN:]
    bd = jnp.kron(jnp.eye(4, dtype=jnp.float32), w1c)
    z = jnp.zeros((64, 128), jnp.float32)
    wa = jnp.concatenate([bd, z], axis=0)
    wb = jnp.concatenate([z, bd], axis=0)
    b4 = jnp.tile(b1, 4).reshape(1, 128)

    G = ea_r.shape[0]
    BG = G // 40
    spec = pl.BlockSpec((BG, 128), lambda i: (i, 0))
    wspec = pl.BlockSpec((128, 128), lambda i: (0, 0))
    cshape = jax.ShapeDtypeStruct((G, 128), jnp.float32)
    return pl.pallas_call(
        _edge_c_body,
        grid=(40,),
        in_specs=[spec, spec, wspec, wspec,
                  pl.BlockSpec((1, 128), lambda i: (0, 0))],
        out_specs=[spec, spec, spec, spec],
        out_shape=[cshape, cshape, cshape, cshape],
    )(ea_r, eat_r, wa, wb, b4)


def _make_sc_kernel(n_edges):
    per_tile = n_edges // NW
    n_chunks = per_tile // CH

    def buf_set():
        return [
            pltpu.VMEM((CH, 2 * H), jnp.float32),
            pltpu.VMEM((CH, 2 * H), jnp.float32),
            pltpu.VMEM((CH // 8, 128), jnp.float32),
            pltpu.VMEM((CH // 8, 128), jnp.float32),
            pltpu.VMEM((CH // 8, 128), jnp.float32),
            pltpu.VMEM((CH // 8, 128), jnp.float32),
        ]

    @functools.partial(
        pl.kernel,
        out_type=jax.ShapeDtypeStruct((n_edges,), jnp.float32),
        mesh=plsc.VectorSubcoreMesh(core_axis_name="c", subcore_axis_name="s"),
        compiler_params=pltpu.CompilerParams(needs_layout_passes=False,
                                             use_tc_tiling_on_sc=False),
        scratch_types=[
            pltpu.VMEM((per_tile,), jnp.int32),
            pltpu.VMEM((per_tile,), jnp.int32),
            *buf_set(),
            *buf_set(),
            pltpu.VMEM((3 * L,), jnp.float32),
            pltpu.VMEM((GRP * L,), jnp.float32),
            pltpu.VMEM((per_tile,), jnp.float32),
            pltpu.SemaphoreType.DMA,
            pltpu.SemaphoreType.DMA,
            pltpu.SemaphoreType.DMA,
            pltpu.SemaphoreType.DMA,
        ],
    )
    def _sc_edge_mlp(ab_hbm, c1a_hbm, c1b_hbm, c2a_hbm, c2b_hbm,
                     src_hbm, tgt_hbm, par_hbm, out_hbm,
                     srcv, tgtv,
                     absv_a, abtv_a, c1av_a, c1bv_a, c2av_a, c2bv_a,
                     absv_b, abtv_b, c1av_b, c1bv_b, c2av_b, c2bv_b,
                     pv, stg, outv,
                     sem_g_a, sem_g_b, sem_c_a, sem_c_b):
        wid = lax.axis_index("s") * NC + lax.axis_index("c")
        base = wid * per_tile

        pltpu.sync_copy(par_hbm, pv)
        pltpu.sync_copy(src_hbm.at[pl.ds(base, per_tile)], srcv)
        pltpu.sync_copy(tgt_hbm.at[pl.ds(base, per_tile)], tgtv)
        wd0 = pv[pl.ds(0, L)]
        wd1 = pv[pl.ds(L, L)]
        cdv = pv[pl.ds(2 * L, L)]
        col = lax.iota(jnp.int32, L) * L

        rings = (
            (absv_a, abtv_a, c1av_a, c1bv_a, c2av_a, c2bv_a, sem_g_a, sem_c_a),
            (absv_b, abtv_b, c1av_b, c1bv_b, c2av_b, c2bv_b, sem_g_b, sem_c_b),
        )

        def dma_set(k, ring):
            absv, abtv, c1av, c1bv, c2av, c2bv, sem_g, sem_c = ring
            koff = k * CH
            gr = (base + koff) // 8
            return (
                pltpu.make_async_copy(ab_hbm.at[srcv.at[pl.ds(koff, CH)]],
                                      absv, sem_g),
                pltpu.make_async_copy(ab_hbm.at[tgtv.at[pl.ds(koff, CH)]],
                                      abtv, sem_g),
                pltpu.make_async_copy(c1a_hbm.at[pl.ds(gr, CH // 8), :],
                                      c1av, sem_c),
                pltpu.make_async_copy(c1b_hbm.at[pl.ds(gr, CH // 8), :],
                                      c1bv, sem_c),
                pltpu.make_async_copy(c2a_hbm.at[pl.ds(gr, CH // 8), :],
                                      c2av, sem_c),
                pltpu.make_async_copy(c2b_hbm.at[pl.ds(gr, CH // 8), :],
                                      c2bv, sem_c),
            )

        def issue(k, ring):
            for cp in dma_set(k, ring):
                cp.start()

        def wait(k, ring):
            for cp in dma_set(k, ring):
                cp.wait()

        def compute(k, ring):
            absv, abtv, c1av, c1bv, c2av, c2bv, _, _ = ring
            koff = k * CH

            def grp_body(blk, carry2):
                e0 = blk * GRP
                for j in range(GRP):
                    e = e0 + j
                    el = j % 8
                    gi = blk * 2 + j // 8
                    if el < 4:
                        r1, r2, co = c1av, c2av, el * 32
                    else:
                        r1, r2, co = c1bv, c2bv, (el - 4) * 32
                    s0 = absv[e, pl.ds(0, L)]
                    s1 = absv[e, pl.ds(L, L)]
                    s2 = absv[e, pl.ds(2 * L, L)]
                    s3 = absv[e, pl.ds(3 * L, L)]
                    t0 = abtv[e, pl.ds(0, L)]
                    t1 = abtv[e, pl.ds(L, L)]
                    t2 = abtv[e, pl.ds(2 * L, L)]
                    t3 = abtv[e, pl.ds(3 * L, L)]
                    c10 = r1[gi, pl.ds(co, L)]
                    c11 = r1[gi, pl.ds(co + L, L)]
                    c20 = r2[gi, pl.ds(co, L)]
                    c21 = r2[gi, pl.ds(co + L, L)]
                    zero = jnp.zeros((L,), jnp.float32)
                    h1a = jnp.maximum(s0 + t2 + c10, zero)
                    h1b = jnp.maximum(s1 + t3 + c11, zero)
                    h2a = jnp.maximum(t0 + s2 + c20, zero)
                    h2b = jnp.maximum(t1 + s3 + c21, zero)
                    t = (h1a + h2a) * wd0 + (h1b + h2b) * wd1
                    stg[pl.ds(j * L, L)] = t
                sums = plsc.load_gather(stg, [col])
                for kk in range(1, L):
                    sums = sums + plsc.load_gather(stg, [col + kk])
                z = sums + cdv
                outv[pl.ds(koff + e0, L)] = 1.0 / (1.0 + jnp.exp(-z))
                return carry2

            lax.fori_loop(0, CH // GRP, grp_body, 0, unroll=False)

        issue(0, rings[0])

        def pair_body(kp, carry):
            k0 = kp * 2
            k1 = k0 + 1

            @pl.when(k1 < n_chunks)
            def _():
                issue(k1, rings[1])

            wait(k0, rings[0])
            compute(k0, rings[0])

            @pl.when(k0 + 2 < n_chunks)
            def _():
                issue(k0 + 2, rings[0])

            @pl.when(k1 < n_chunks)
            def _():
                wait(k1, rings[1])
                compute(k1, rings[1])

            return carry

        lax.fori_loop(0, (n_chunks + 1) // 2, pair_body, 0, unroll=False)

        pltpu.sync_copy(outv, out_hbm.at[pl.ds(base, per_tile)])

    return _sc_edge_mlp


_sc_half0 = _make_sc_kernel(E0)
_sc_half1 = _make_sc_kernel(E1)


def kernel(x, edge_index, edge_attr, edge_attr_T, W1, b1, W2, b2):
    ab = _ab_table(x, W1)
    wd = 0.5 * (W2[:, 1] - W2[:, 0])
    cd = b2[1] - b2[0]
    params = jnp.concatenate([wd, jnp.full((L,), cd, jnp.float32)])
    src = edge_index[0].astype(jnp.int32)
    tgt = edge_index[1].astype(jnp.int32)

    outs = []
    for lo, hi, sck in ((0, E0, _sc_half0), (E0, E, _sc_half1)):
        ea_r = edge_attr[lo:hi].reshape((hi - lo) // 8, 128)
        eat_r = edge_attr_T[lo:hi].reshape((hi - lo) // 8, 128)
        c1a, c1b, c2a, c2b = _edge_c(ea_r, eat_r, W1, b1)
        outs.append(sck(ab, c1a, c1b, c2a, c2b,
                        src[lo:hi], tgt[lo:hi], params))
    return jnp.concatenate(outs).reshape(E, 1)
